# baseline (speedup 1.0000x reference)
import functools

import jax
import jax.numpy as jnp
from jax import lax
from jax.experimental import pallas as pl
from jax.experimental.pallas import tpu as pltpu
from jax.experimental.pallas import tpu_sc as plsc

B = 16384
D = 64
NC = 2
NS = 16
NW = NC * NS
BPW = B // NW
IDX_ROWS = BPW // 128
GROUPS = BPW // 16

_mesh = plsc.VectorSubcoreMesh(
    core_axis_name="c", subcore_axis_name="s", num_cores=NC, num_subcores=NS
)


@functools.partial(
    pl.kernel,
    out_type=jax.ShapeDtypeStruct((B,), jnp.float32),
    mesh=_mesh,
    scratch_types=[
        pltpu.VMEM((IDX_ROWS, 128), jnp.int32),
        pltpu.VMEM((IDX_ROWS, 128), jnp.int32),
        pltpu.VMEM((BPW,), jnp.float32),
        pltpu.VMEM((BPW,), jnp.float32),
        pltpu.VMEM((BPW * D,), jnp.float32),
        pltpu.VMEM((BPW * D,), jnp.float32),
        pltpu.VMEM((D,), jnp.float32),
        pltpu.VMEM((16,), jnp.float32),
        pltpu.VMEM((16, 16), jnp.float32),
        pltpu.VMEM((BPW,), jnp.float32),
        pltpu.SemaphoreType.DMA,
        pltpu.SemaphoreType.DMA,
    ],
    compiler_params=pltpu.CompilerParams(needs_layout_passes=False),
)
def _sc_kernel(
    uf_hbm, if_hbm, uid_hbm, iid_hbm, w_hbm, b_hbm, bu_hbm, bi_hbm,
    out_hbm,
    idx_u, idx_i, bu_v, bi_v, uf_v, if_v, w_v, b_v, p_v, out_v,
    sem_g, sem_f,
):
    wid = lax.axis_index("s") * NC + lax.axis_index("c")
    base = pl.multiple_of(wid * BPW, BPW)
    irow = wid * IDX_ROWS

    pltpu.sync_copy(uid_hbm.at[pl.ds(irow, IDX_ROWS)], idx_u)
    pltpu.sync_copy(iid_hbm.at[pl.ds(irow, IDX_ROWS)], idx_i)
    pltpu.sync_copy(w_hbm, w_v)
    pltpu.sync_copy(b_hbm, b_v)

    gathers = []
    for j in range(IDX_ROWS):
        gathers.append(
            pltpu.async_copy(bu_hbm.at[idx_u.at[j]], bu_v.at[pl.ds(j * 128, 128)], sem_g)
        )
        gathers.append(
            pltpu.async_copy(bi_hbm.at[idx_i.at[j]], bi_v.at[pl.ds(j * 128, 128)], sem_g)
        )

    fbase = pl.multiple_of(wid * (BPW * D), BPW * D)
    cp_u = pltpu.async_copy(uf_hbm.at[pl.ds(fbase, BPW * D)], uf_v, sem_f)
    cp_i = pltpu.async_copy(if_hbm.at[pl.ds(fbase, BPW * D)], if_v, sem_f)

    for g in gathers:
        g.wait()
    cp_u.wait()
    cp_i.wait()

    w_regs = [w_v[pl.ds(16 * j, 16)] for j in range(D // 16)]
    b_vec = b_v[...]
    iota16 = lax.iota(jnp.int32, 16)

    def group_body(g, carry):
        rbase = pl.multiple_of(g * 16, 16)
        gflat = pl.multiple_of(g * 16 * D, 16 * D)
        for r in range(16):
            acc = (
                uf_v[pl.ds(gflat + r * D, 16)]
                * if_v[pl.ds(gflat + r * D, 16)]
                * w_regs[0]
            )
            for j in range(1, D // 16):
                acc = acc + (
                    uf_v[pl.ds(gflat + r * D + 16 * j, 16)]
                    * if_v[pl.ds(gflat + r * D + 16 * j, 16)]
                    * w_regs[j]
                )
            p_v[r] = acc
        res = b_vec + bu_v[pl.ds(rbase, 16)] + bi_v[pl.ds(rbase, 16)]
        for c in range(16):
            res = res + plsc.load_gather(
                p_v, [iota16, jnp.full((16,), c, jnp.int32)]
            )
        out_v[pl.ds(rbase, 16)] = res
        return carry

    lax.fori_loop(0, GROUPS, group_body, 0)

    pltpu.sync_copy(out_v, out_hbm.at[pl.ds(base, BPW)])


def kernel(user_feature, user_id, item_feature, item_id, W, b, b_user, b_item):
    uid = user_id.reshape(B // 128, 128).astype(jnp.int32)
    iid = item_id.reshape(B // 128, 128).astype(jnp.int32)
    w_flat = W.reshape(D)
    b16 = jnp.broadcast_to(b, (16,))
    out = _sc_kernel(
        user_feature.reshape(B * D), item_feature.reshape(B * D),
        uid, iid, w_flat, b16, b_user, b_item
    )
    return out.reshape(B, 1)


# speedup vs baseline: 1.0398x; 1.0398x over previous
import functools

import jax
import jax.numpy as jnp
from jax import lax
from jax.experimental import pallas as pl
from jax.experimental.pallas import tpu as pltpu
from jax.experimental.pallas import tpu_sc as plsc

B = 16384
D = 64
NC = 2
NS = 16
NW = NC * NS
BPW = B // NW
FPW = BPW * D
IDX_ROWS = BPW // 128
GROUPS = BPW // 16
HGROUPS = GROUPS // 2

_mesh = plsc.VectorSubcoreMesh(
    core_axis_name="c", subcore_axis_name="s", num_cores=NC, num_subcores=NS
)


@functools.partial(
    pl.kernel,
    out_type=jax.ShapeDtypeStruct((B,), jnp.float32),
    mesh=_mesh,
    scratch_types=[
        pltpu.VMEM((2 * IDX_ROWS, 128), jnp.int32),
        pltpu.VMEM((2 * BPW,), jnp.float32),
        pltpu.VMEM((2 * FPW,), jnp.float32),
        pltpu.VMEM((D + 16,), jnp.float32),
        pltpu.VMEM((BPW * 16,), jnp.float32),
        pltpu.VMEM((BPW,), jnp.float32),
        pltpu.SemaphoreType.DMA,
        pltpu.SemaphoreType.DMA,
        pltpu.SemaphoreType.DMA,
    ],
    compiler_params=pltpu.CompilerParams(needs_layout_passes=False),
)
def _sc_kernel(
    uf_hbm, if_hbm, ids_hbm, wb_hbm, bu_hbm, bi_hbm,
    out_hbm,
    idx_v, bias_v, feat_v, wb_v, p_v, out_v,
    sem_g, sem_f0, sem_f1,
):
    wid = lax.axis_index("s") * NC + lax.axis_index("c")
    base = pl.multiple_of(wid * BPW, BPW)
    irow = wid * IDX_ROWS

    pltpu.sync_copy(ids_hbm.at[pl.ds(irow, IDX_ROWS)], idx_v.at[pl.ds(0, IDX_ROWS)])
    pltpu.sync_copy(
        ids_hbm.at[pl.ds((B // 128) + irow, IDX_ROWS)],
        idx_v.at[pl.ds(IDX_ROWS, IDX_ROWS)],
    )
    pltpu.sync_copy(wb_hbm, wb_v)

    gathers = []
    for j in range(IDX_ROWS):
        gathers.append(
            pltpu.async_copy(
                bu_hbm.at[idx_v.at[j]], bias_v.at[pl.ds(j * 128, 128)], sem_g
            )
        )
        gathers.append(
            pltpu.async_copy(
                bi_hbm.at[idx_v.at[IDX_ROWS + j]],
                bias_v.at[pl.ds(BPW + j * 128, 128)],
                sem_g,
            )
        )

    fbase = pl.multiple_of(wid * FPW, FPW)
    half = FPW // 2
    cp0u = pltpu.async_copy(
        uf_hbm.at[pl.ds(fbase, half)], feat_v.at[pl.ds(0, half)], sem_f0
    )
    cp0i = pltpu.async_copy(
        if_hbm.at[pl.ds(fbase, half)], feat_v.at[pl.ds(FPW, half)], sem_f0
    )
    cp1u = pltpu.async_copy(
        uf_hbm.at[pl.ds(fbase + half, half)], feat_v.at[pl.ds(half, half)], sem_f1
    )
    cp1i = pltpu.async_copy(
        if_hbm.at[pl.ds(fbase + half, half)],
        feat_v.at[pl.ds(FPW + half, half)],
        sem_f1,
    )

    w_regs = [wb_v[pl.ds(16 * j, 16)] for j in range(D // 16)]
    b_vec = wb_v[pl.ds(D, 16)]
    iota16 = lax.iota(jnp.int32, 16)

    def rows_body(g, carry):
        gflat = pl.multiple_of(g * 16 * D, 16 * D)
        gp = pl.multiple_of(g * 256, 256)
        for r in range(16):
            u0 = pl.multiple_of(gflat + r * D, 16)
            acc = (
                feat_v[pl.ds(u0, 16)] * feat_v[pl.ds(FPW + u0, 16)] * w_regs[0]
            )
            for j in range(1, D // 16):
                acc = acc + (
                    feat_v[pl.ds(u0 + 16 * j, 16)]
                    * feat_v[pl.ds(FPW + u0 + 16 * j, 16)]
                    * w_regs[j]
                )
            p_v[pl.ds(gp + r * 16, 16)] = acc
        return carry

    cp0u.wait()
    cp0i.wait()
    lax.fori_loop(0, HGROUPS, rows_body, 0)
    cp1u.wait()
    cp1i.wait()
    lax.fori_loop(HGROUPS, GROUPS, rows_body, 0)

    for g in gathers:
        g.wait()

    pidx = iota16 * 16

    def red_body(g, carry):
        gp = pl.multiple_of(g * 256, 256)
        rbase = pl.multiple_of(g * 16, 16)
        bidx = pidx + gp
        s = [
            plsc.load_gather(p_v, [bidx + c]) for c in range(4)
        ]
        for c in range(4, 16):
            s[c % 4] = s[c % 4] + plsc.load_gather(p_v, [bidx + c])
        res = (
            (b_vec + bias_v[pl.ds(rbase, 16)])
            + (bias_v[pl.ds(BPW + rbase, 16)] + s[0])
            + (s[1] + s[2])
            + s[3]
        )
        out_v[pl.ds(rbase, 16)] = res
        return carry

    lax.fori_loop(0, GROUPS, red_body, 0)

    pltpu.sync_copy(out_v, out_hbm.at[pl.ds(base, BPW)])


def kernel(user_feature, user_id, item_feature, item_id, W, b, b_user, b_item):
    ids = jnp.concatenate(
        [
            user_id.astype(jnp.int32).reshape(B // 128, 128),
            item_id.astype(jnp.int32).reshape(B // 128, 128),
        ]
    )
    wb = jnp.concatenate([W.reshape(D), jnp.broadcast_to(b, (16,))])
    out = _sc_kernel(
        user_feature.reshape(B * D),
        item_feature.reshape(B * D),
        ids,
        wb,
        b_user,
        b_item,
    )
    return out.reshape(B, 1)


# speedup vs baseline: 1.5009x; 1.4435x over previous
import functools

import jax
import jax.numpy as jnp
from jax import lax
from jax.experimental import pallas as pl
from jax.experimental.pallas import tpu as pltpu
from jax.experimental.pallas import tpu_sc as plsc

B = 16384
D = 64
NC = 2
NS = 16
NW = NC * NS
BPW = B // NW
IDX_ROWS = BPW // 128

_mesh = plsc.VectorSubcoreMesh(
    core_axis_name="c", subcore_axis_name="s", num_cores=NC, num_subcores=NS
)


@functools.partial(
    pl.kernel,
    out_type=jax.ShapeDtypeStruct((B // 128, 128), jnp.float32),
    mesh=_mesh,
    scratch_types=[
        pltpu.VMEM((IDX_ROWS, 128), jnp.int32),
        pltpu.VMEM((IDX_ROWS, 128), jnp.int32),
        pltpu.VMEM((IDX_ROWS, 128), jnp.float32),
        pltpu.VMEM((IDX_ROWS, 128), jnp.float32),
        pltpu.SemaphoreType.DMA,
    ],
    compiler_params=pltpu.CompilerParams(needs_layout_passes=False),
)
def _sc_bias_kernel(
    uid_hbm, iid_hbm, bu_hbm, bi_hbm,
    out_hbm,
    idx_u, idx_i, bu_v, bi_v, sem,
):
    wid = lax.axis_index("s") * NC + lax.axis_index("c")
    irow = wid * IDX_ROWS

    pltpu.sync_copy(uid_hbm.at[pl.ds(irow, IDX_ROWS)], idx_u)
    pltpu.sync_copy(iid_hbm.at[pl.ds(irow, IDX_ROWS)], idx_i)

    gathers = []
    for j in range(IDX_ROWS):
        gathers.append(pltpu.async_copy(bu_hbm.at[idx_u.at[j]], bu_v.at[j], sem))
        gathers.append(pltpu.async_copy(bi_hbm.at[idx_i.at[j]], bi_v.at[j], sem))
    for g in gathers:
        g.wait()

    for j in range(IDX_ROWS):
        for k in range(128 // 16):
            s = bu_v[j, pl.ds(16 * k, 16)] + bi_v[j, pl.ds(16 * k, 16)]
            bu_v[j, pl.ds(16 * k, 16)] = s

    pltpu.sync_copy(bu_v, out_hbm.at[pl.ds(irow, IDX_ROWS)])


_TC_BLK = 2048


def _tc_dense_body(uf_ref, if_ref, w_ref, b_ref, o_ref):
    t = uf_ref[...] * if_ref[...]
    s = jnp.sum(t * w_ref[...], axis=1) + b_ref[0]
    o_ref[...] = s.reshape(_TC_BLK // 128, 128)


_tc_dense = pl.pallas_call(
    _tc_dense_body,
    grid=(B // _TC_BLK,),
    in_specs=[
        pl.BlockSpec((_TC_BLK, D), lambda i: (i, 0)),
        pl.BlockSpec((_TC_BLK, D), lambda i: (i, 0)),
        pl.BlockSpec((1, D), lambda i: (0, 0)),
        pl.BlockSpec(memory_space=pltpu.SMEM),
    ],
    out_specs=pl.BlockSpec((_TC_BLK // 128, 128), lambda i: (i, 0)),
    out_shape=jax.ShapeDtypeStruct((B // 128, 128), jnp.float32),
)


def kernel(user_feature, user_id, item_feature, item_id, W, b, b_user, b_item):
    uid = user_id.astype(jnp.int32).reshape(B // 128, 128)
    iid = item_id.astype(jnp.int32).reshape(B // 128, 128)
    bias = _sc_bias_kernel(uid, iid, b_user, b_item)
    lin = _tc_dense(user_feature, item_feature, W, b)
    return (lin + bias).reshape(B, 1)


# speedup vs baseline: 2.4056x; 1.6028x over previous
import functools

import jax
import jax.numpy as jnp
from jax import lax
from jax.experimental import pallas as pl
from jax.experimental.pallas import tpu as pltpu
from jax.experimental.pallas import tpu_sc as plsc

B = 16384
D = 64
NC = 2
NS = 16
NW = NC * NS
BPW = B // NW
IDX_ROWS = BPW // 128

_mesh = plsc.VectorSubcoreMesh(
    core_axis_name="c", subcore_axis_name="s", num_cores=NC, num_subcores=NS
)


@functools.partial(
    pl.kernel,
    out_type=jax.ShapeDtypeStruct((B // 128, 128), jnp.float32),
    mesh=_mesh,
    scratch_types=[
        pltpu.VMEM((IDX_ROWS, 128), jnp.int32),
        pltpu.VMEM((IDX_ROWS, 128), jnp.int32),
        pltpu.VMEM((IDX_ROWS, 128), jnp.float32),
        pltpu.VMEM((IDX_ROWS, 128), jnp.float32),
        pltpu.SemaphoreType.DMA,
    ],
    compiler_params=pltpu.CompilerParams(needs_layout_passes=False),
)
def _sc_bias_kernel(
    uid_hbm, iid_hbm, bu_hbm, bi_hbm,
    out_hbm,
    idx_u, idx_i, bu_v, bi_v, sem,
):
    wid = lax.axis_index("s") * NC + lax.axis_index("c")
    irow = wid * IDX_ROWS

    pltpu.sync_copy(uid_hbm.at[pl.ds(irow, IDX_ROWS)], idx_u)
    pltpu.sync_copy(iid_hbm.at[pl.ds(irow, IDX_ROWS)], idx_i)

    gathers = []
    for j in range(IDX_ROWS):
        gathers.append(pltpu.async_copy(bu_hbm.at[idx_u.at[j]], bu_v.at[j], sem))
        gathers.append(pltpu.async_copy(bi_hbm.at[idx_i.at[j]], bi_v.at[j], sem))
    for g in gathers:
        g.wait()

    for j in range(IDX_ROWS):
        for k in range(128 // 16):
            s = bu_v[j, pl.ds(16 * k, 16)] + bi_v[j, pl.ds(16 * k, 16)]
            bu_v[j, pl.ds(16 * k, 16)] = s

    pltpu.sync_copy(bu_v, out_hbm.at[pl.ds(irow, IDX_ROWS)])


_TC_BLK = 2048


def _tc_dense_body(uft_ref, ift_ref, w_ref, b_ref, o_ref):
    tw = uft_ref[...] * ift_ref[...] * w_ref[...]
    s = jnp.sum(tw, axis=0) + b_ref[0]
    o_ref[...] = s.reshape(_TC_BLK // 128, 128)


_tc_dense = pl.pallas_call(
    _tc_dense_body,
    grid=(B // _TC_BLK,),
    in_specs=[
        pl.BlockSpec((D, _TC_BLK), lambda i: (0, i)),
        pl.BlockSpec((D, _TC_BLK), lambda i: (0, i)),
        pl.BlockSpec((D, 1), lambda i: (0, 0)),
        pl.BlockSpec(memory_space=pltpu.SMEM),
    ],
    out_specs=pl.BlockSpec((_TC_BLK // 128, 128), lambda i: (i, 0)),
    out_shape=jax.ShapeDtypeStruct((B // 128, 128), jnp.float32),
)


def kernel(user_feature, user_id, item_feature, item_id, W, b, b_user, b_item):
    uid = user_id.astype(jnp.int32).reshape(B // 128, 128)
    iid = item_id.astype(jnp.int32).reshape(B // 128, 128)
    bias = _sc_bias_kernel(uid, iid, b_user, b_item)
    lin = _tc_dense(user_feature.T, item_feature.T, W.T, b)
    return (lin + bias).reshape(B, 1)


# speedup vs baseline: 2.4887x; 1.0345x over previous
import functools

import jax
import jax.numpy as jnp
from jax import lax
from jax.experimental import pallas as pl
from jax.experimental.pallas import tpu as pltpu
from jax.experimental.pallas import tpu_sc as plsc

B = 16384
D = 64
NC = 2
NS = 16
NW = NC * NS
BPW = B // NW
IDX_ROWS = BPW // 128

_mesh = plsc.VectorSubcoreMesh(
    core_axis_name="c", subcore_axis_name="s", num_cores=NC, num_subcores=NS
)


@functools.partial(
    pl.kernel,
    out_type=jax.ShapeDtypeStruct((B // 128, 128), jnp.float32),
    mesh=_mesh,
    scratch_types=[
        pltpu.VMEM((IDX_ROWS, 128), jnp.int32),
        pltpu.VMEM((IDX_ROWS, 128), jnp.int32),
        pltpu.VMEM((IDX_ROWS, 128), jnp.float32),
        pltpu.VMEM((IDX_ROWS, 128), jnp.float32),
        pltpu.SemaphoreType.DMA,
    ],
    compiler_params=pltpu.CompilerParams(needs_layout_passes=False),
)
def _sc_bias_kernel(
    uid_hbm, iid_hbm, bu_hbm, bi_hbm,
    out_hbm,
    idx_u, idx_i, bu_v, bi_v, sem,
):
    wid = lax.axis_index("s") * NC + lax.axis_index("c")
    irow = wid * IDX_ROWS

    pltpu.sync_copy(uid_hbm.at[pl.ds(irow, IDX_ROWS)], idx_u)
    pltpu.sync_copy(iid_hbm.at[pl.ds(irow, IDX_ROWS)], idx_i)

    gathers = []
    for j in range(IDX_ROWS):
        gathers.append(pltpu.async_copy(bu_hbm.at[idx_u.at[j]], bu_v.at[j], sem))
        gathers.append(pltpu.async_copy(bi_hbm.at[idx_i.at[j]], bi_v.at[j], sem))
    for g in gathers:
        g.wait()

    for j in range(IDX_ROWS):
        for k in range(128 // 16):
            s = bu_v[j, pl.ds(16 * k, 16)] + bi_v[j, pl.ds(16 * k, 16)]
            bu_v[j, pl.ds(16 * k, 16)] = s

    pltpu.sync_copy(bu_v, out_hbm.at[pl.ds(irow, IDX_ROWS)])


_TC_BLK = 4096


def _tc_dense_body(uft_ref, ift_ref, w_ref, b_ref, o_ref):
    t = uft_ref[...] * ift_ref[...]
    s = jax.lax.dot_general(
        w_ref[...], t, (((1,), (0,)), ((), ())),
        preferred_element_type=jnp.float32,
    )
    o_ref[...] = (s + b_ref[0]).reshape(_TC_BLK // 128, 128)


_tc_dense = pl.pallas_call(
    _tc_dense_body,
    grid=(B // _TC_BLK,),
    in_specs=[
        pl.BlockSpec((D, _TC_BLK), lambda i: (0, i)),
        pl.BlockSpec((D, _TC_BLK), lambda i: (0, i)),
        pl.BlockSpec((1, D), lambda i: (0, 0)),
        pl.BlockSpec(memory_space=pltpu.SMEM),
    ],
    out_specs=pl.BlockSpec((_TC_BLK // 128, 128), lambda i: (i, 0)),
    out_shape=jax.ShapeDtypeStruct((B // 128, 128), jnp.float32),
)


def kernel(user_feature, user_id, item_feature, item_id, W, b, b_user, b_item):
    uid = user_id.astype(jnp.int32).reshape(B // 128, 128)
    iid = item_id.astype(jnp.int32).reshape(B // 128, 128)
    bias = _sc_bias_kernel(uid, iid, b_user, b_item)
    lin = _tc_dense(user_feature.T, item_feature.T, W, b)
    return (lin + bias).reshape(B, 1)


# speedup vs baseline: 2.5457x; 1.0229x over previous
import functools

import jax
import jax.numpy as jnp
from jax import lax
from jax.experimental import pallas as pl
from jax.experimental.pallas import tpu as pltpu
from jax.experimental.pallas import tpu_sc as plsc

B = 16384
D = 64
NC = 2
NS = 16
NW = NC * NS
BPW = B // NW
IDX_ROWS = BPW // 128

_mesh = plsc.VectorSubcoreMesh(
    core_axis_name="c", subcore_axis_name="s", num_cores=NC, num_subcores=NS
)


@functools.partial(
    pl.kernel,
    out_type=jax.ShapeDtypeStruct((B // 128, 128), jnp.float32),
    mesh=_mesh,
    scratch_types=[
        pltpu.VMEM((IDX_ROWS, 128), jnp.int32),
        pltpu.VMEM((IDX_ROWS, 128), jnp.int32),
        pltpu.VMEM((IDX_ROWS, 128), jnp.float32),
        pltpu.VMEM((IDX_ROWS, 128), jnp.float32),
        pltpu.SemaphoreType.DMA,
    ],
    compiler_params=pltpu.CompilerParams(needs_layout_passes=False),
)
def _sc_bias_kernel(
    uid_hbm, iid_hbm, bu_hbm, bi_hbm,
    out_hbm,
    idx_u, idx_i, bu_v, bi_v, sem,
):
    wid = lax.axis_index("s") * NC + lax.axis_index("c")
    irow = wid * IDX_ROWS

    cp_u = pltpu.async_copy(uid_hbm.at[pl.ds(irow, IDX_ROWS)], idx_u, sem)
    cp_i = pltpu.async_copy(iid_hbm.at[pl.ds(irow, IDX_ROWS)], idx_i, sem)
    cp_u.wait()
    cp_i.wait()

    gathers = []
    for j in range(IDX_ROWS):
        gathers.append(pltpu.async_copy(bu_hbm.at[idx_u.at[j]], bu_v.at[j], sem))
        gathers.append(pltpu.async_copy(bi_hbm.at[idx_i.at[j]], bi_v.at[j], sem))
    for g in gathers:
        g.wait()

    for j in range(IDX_ROWS):
        for k in range(128 // 16):
            s = bu_v[j, pl.ds(16 * k, 16)] + bi_v[j, pl.ds(16 * k, 16)]
            bu_v[j, pl.ds(16 * k, 16)] = s

    pltpu.sync_copy(bu_v, out_hbm.at[pl.ds(irow, IDX_ROWS)])


_TC_BLK = 4096


def _tc_dense_body(uft_ref, ift_ref, w_ref, b_ref, o_ref):
    t = uft_ref[...] * ift_ref[...]
    s = jax.lax.dot_general(
        w_ref[...], t, (((1,), (0,)), ((), ())),
        preferred_element_type=jnp.float32,
    )
    o_ref[...] = (s + b_ref[0]).reshape(_TC_BLK // 128, 128)


_tc_dense = pl.pallas_call(
    _tc_dense_body,
    grid=(B // _TC_BLK,),
    in_specs=[
        pl.BlockSpec((D, _TC_BLK), lambda i: (0, i)),
        pl.BlockSpec((D, _TC_BLK), lambda i: (0, i)),
        pl.BlockSpec((1, D), lambda i: (0, 0)),
        pl.BlockSpec(memory_space=pltpu.SMEM),
    ],
    out_specs=pl.BlockSpec((_TC_BLK // 128, 128), lambda i: (i, 0)),
    out_shape=jax.ShapeDtypeStruct((B // 128, 128), jnp.float32),
)


def kernel(user_feature, user_id, item_feature, item_id, W, b, b_user, b_item):
    uid = user_id.astype(jnp.int32).reshape(B // 128, 128)
    iid = item_id.astype(jnp.int32).reshape(B // 128, 128)
    bias = _sc_bias_kernel(uid, iid, b_user, b_item)
    lin = _tc_dense(user_feature.T, item_feature.T, W, b)
    return (lin + bias).reshape(B, 1)
